# baseline (speedup 1.0000x reference)
import functools

import jax
import jax.numpy as jnp
from jax import lax
from jax.experimental import pallas as pl
from jax.experimental.pallas import tpu as pltpu
from jax.experimental.pallas import tpu_sc as plsc

_LANES = 16


def _full16(v):
    return jnp.full((_LANES,), v, dtype=jnp.int32)


def sc_stream_add(x_flat, step_idx, label_idx, step_table, label_table,
                  batch, embed, seq):
    info = plsc.get_sparse_core_info()
    num_cores = info.num_cores
    nw = info.num_cores * info.num_subcores
    b_per_w = batch // nw
    rows_per_chunk = 32
    chunk = rows_per_chunk * seq
    chunks_per_b = embed // rows_per_chunk
    n_chunks = b_per_w * chunks_per_b
    slab = embed * seq
    mesh = plsc.VectorSubcoreMesh(core_axis_name="c", subcore_axis_name="s")

    @functools.partial(
        pl.kernel,
        mesh=mesh,
        out_type=jax.ShapeDtypeStruct((batch * embed * seq,), jnp.float32),
        scratch_types=[
            pltpu.VMEM((b_per_w,), jnp.int32),
            pltpu.VMEM((b_per_w, embed), jnp.float32),
            pltpu.VMEM((b_per_w,), jnp.int32),
            pltpu.VMEM((b_per_w, embed), jnp.float32),
            pltpu.VMEM((b_per_w * embed,), jnp.float32),
            pltpu.VMEM((chunk,), jnp.float32),
            pltpu.VMEM((chunk,), jnp.float32),
            pltpu.VMEM((chunk,), jnp.float32),
            pltpu.VMEM((chunk,), jnp.float32),
            pltpu.SemaphoreType.DMA,
            pltpu.SemaphoreType.DMA,
            pltpu.SemaphoreType.DMA,
            pltpu.SemaphoreType.DMA,
            pltpu.SemaphoreType.DMA,
            pltpu.SemaphoreType.DMA,
            pltpu.SemaphoreType.DMA,
        ],
    )
    def k(step_idx_hbm, label_idx_hbm, step_tab_hbm, label_tab_hbm, x_hbm,
          out_hbm,
          sidx_v, srow_v, lidx_v, lrow_v, comb_v, in0, in1, ou0, ou1,
          isem, ssem, lsem, gsem0, gsem1, wsem0, wsem1):
        wid = lax.axis_index("s") * num_cores + lax.axis_index("c")
        base_b = wid * b_per_w
        base_e = base_b * slab

        icp1 = pltpu.async_copy(step_idx_hbm.at[pl.ds(base_b, b_per_w)], sidx_v, isem)
        icp2 = pltpu.async_copy(label_idx_hbm.at[pl.ds(base_b, b_per_w)], lidx_v, isem)
        icp1.wait()
        icp2.wait()
        scp = pltpu.async_copy(step_tab_hbm.at[sidx_v], srow_v, ssem)
        lcp = pltpu.async_copy(label_tab_hbm.at[lidx_v], lrow_v, lsem)
        scp.wait()
        lcp.wait()
        for r in range(b_per_w):
            for j in range(embed // _LANES):
                sl = pl.ds(j * _LANES, _LANES)
                comb_v[pl.ds(r * embed + j * _LANES, _LANES)] = (
                    srow_v[r, sl] + lrow_v[r, sl])

        in_bufs = (in0, in1)
        ou_bufs = (ou0, ou1)
        gsems = (gsem0, gsem1)
        wsems = (wsem0, wsem1)

        def chunk_off(kk):
            return base_e + (kk >> 2) * slab + (kk & 3) * chunk

        def gather_start(kk, p):
            pltpu.async_copy(x_hbm.at[pl.ds(chunk_off(kk), chunk)],
                             in_bufs[p], gsems[p])

        def gather_wait(p):
            pltpu.make_async_copy(x_hbm.at[pl.ds(0, chunk)],
                                  in_bufs[p], gsems[p]).wait()

        def scatter_start(kk, p):
            pltpu.async_copy(ou_bufs[p],
                             out_hbm.at[pl.ds(chunk_off(kk), chunk)], wsems[p])

        def scatter_wait(p):
            pltpu.make_async_copy(ou_bufs[p], out_hbm.at[pl.ds(0, chunk)],
                                  wsems[p]).wait()

        gather_start(0, 0)
        gather_start(1, 1)

        def compute(kk, p):
            bl = kk >> 2
            d_base = (kk & 3) * rows_per_chunk
            ibuf = in_bufs[p]
            obuf = ou_bufs[p]

            for seg in range(rows_per_chunk // _LANES):
                rv = comb_v[pl.ds(bl * embed + d_base + seg * _LANES, _LANES)]

                def t_body(t, _, rv=rv, seg=seg):
                    splat = rv.at[jnp.full((_LANES,), t, jnp.int32)].get(
                        mode="promise_in_bounds")
                    off = (seg * _LANES + t) * seq
                    for j in range(seq // _LANES):
                        sl = pl.ds(off + j * _LANES, _LANES)
                        obuf[sl] = ibuf[sl] + splat
                    return 0

                lax.fori_loop(0, _LANES, t_body, 0)

        def pair_body(g, _):
            for p in (0, 1):
                kk = 2 * g + p
                gather_wait(p)

                @pl.when(kk >= 2)
                def _():
                    scatter_wait(p)

                compute(kk, p)
                scatter_start(kk, p)

                @pl.when(kk + 2 < n_chunks)
                def _():
                    gather_start(kk + 2, p)
            return 0

        lax.fori_loop(0, n_chunks // 2, pair_body, 0)
        scatter_wait(0)
        scatter_wait(1)

    return k(step_idx, label_idx, step_table, label_table, x_flat)


def kernel(x, step, label, step_table, label_table):
    batch, embed, seq = x.shape
    out_flat = sc_stream_add(
        x.reshape(-1),
        step.reshape(batch).astype(jnp.int32),
        label.reshape(batch).astype(jnp.int32),
        step_table, label_table, batch, embed, seq)
    return out_flat.reshape(batch, embed, seq)


# speedup vs baseline: 5.8065x; 5.8065x over previous
import functools

import jax
import jax.numpy as jnp
from jax import lax
from jax.experimental import pallas as pl
from jax.experimental.pallas import tpu as pltpu
from jax.experimental.pallas import tpu_sc as plsc


def _gather_rows_sc(step_idx, label_idx, step_table, label_table):
    batch = step_idx.shape[0]
    embed = step_table.shape[1]
    info = plsc.get_sparse_core_info()
    num_cores = info.num_cores
    nw = info.num_cores * info.num_subcores
    b_per_w = batch // nw
    mesh = plsc.VectorSubcoreMesh(core_axis_name="c", subcore_axis_name="s")

    @functools.partial(
        pl.kernel,
        mesh=mesh,
        out_type=[
            jax.ShapeDtypeStruct((batch, embed), jnp.float32),
            jax.ShapeDtypeStruct((batch, embed), jnp.float32),
        ],
        scratch_types=[
            pltpu.VMEM((b_per_w,), jnp.int32),
            pltpu.VMEM((b_per_w, embed), jnp.float32),
            pltpu.VMEM((b_per_w,), jnp.int32),
            pltpu.VMEM((b_per_w, embed), jnp.float32),
            pltpu.SemaphoreType.DMA,
            pltpu.SemaphoreType.DMA,
            pltpu.SemaphoreType.DMA,
            pltpu.SemaphoreType.DMA,
        ],
    )
    def gather_kernel(step_idx_hbm, label_idx_hbm, step_tab_hbm, label_tab_hbm,
                      srow_hbm, lrow_hbm,
                      sidx_v, srow_v, lidx_v, lrow_v, isem, ssem, lsem, osem):
        wid = lax.axis_index("s") * num_cores + lax.axis_index("c")
        base = wid * b_per_w
        icp1 = pltpu.async_copy(step_idx_hbm.at[pl.ds(base, b_per_w)], sidx_v, isem)
        icp2 = pltpu.async_copy(label_idx_hbm.at[pl.ds(base, b_per_w)], lidx_v, isem)
        icp1.wait()
        icp2.wait()
        scp = pltpu.async_copy(step_tab_hbm.at[sidx_v], srow_v, ssem)
        lcp = pltpu.async_copy(label_tab_hbm.at[lidx_v], lrow_v, lsem)
        scp.wait()
        ocp1 = pltpu.async_copy(srow_v, srow_hbm.at[pl.ds(base, b_per_w)], osem)
        lcp.wait()
        ocp2 = pltpu.async_copy(lrow_v, lrow_hbm.at[pl.ds(base, b_per_w)], osem)
        ocp1.wait()
        ocp2.wait()

    return gather_kernel(step_idx, label_idx, step_table, label_table)


def _add_body(x_ref, s_ref, l_ref, o_ref):
    emb = s_ref[...] + l_ref[...]
    o_ref[...] = x_ref[...] + emb[:, :, None]


def kernel(x, step, label, step_table, label_table):
    batch, embed, seq = x.shape
    srows, lrows = _gather_rows_sc(
        step.reshape(batch).astype(jnp.int32),
        label.reshape(batch).astype(jnp.int32),
        step_table, label_table)
    bt = 32
    return pl.pallas_call(
        _add_body,
        grid=(batch // bt,),
        in_specs=[
            pl.BlockSpec((bt, embed, seq), lambda i: (i, 0, 0)),
            pl.BlockSpec((bt, embed), lambda i: (i, 0)),
            pl.BlockSpec((bt, embed), lambda i: (i, 0)),
        ],
        out_specs=pl.BlockSpec((bt, embed, seq), lambda i: (i, 0, 0)),
        out_shape=jax.ShapeDtypeStruct((batch, embed, seq), jnp.float32),
    )(x, srows, lrows)


# speedup vs baseline: 5.8091x; 1.0005x over previous
import functools

import jax
import jax.numpy as jnp
from jax import lax
from jax.experimental import pallas as pl
from jax.experimental.pallas import tpu as pltpu
from jax.experimental.pallas import tpu_sc as plsc


def _gather_rows_sc(step_idx, label_idx, step_table, label_table):
    batch = step_idx.shape[0]
    embed = step_table.shape[1]
    info = plsc.get_sparse_core_info()
    num_cores = info.num_cores
    nw = info.num_cores * info.num_subcores
    b_per_w = batch // nw
    mesh = plsc.VectorSubcoreMesh(core_axis_name="c", subcore_axis_name="s")

    @functools.partial(
        pl.kernel,
        mesh=mesh,
        out_type=[
            jax.ShapeDtypeStruct((batch, embed), jnp.float32),
            jax.ShapeDtypeStruct((batch, embed), jnp.float32),
        ],
        scratch_types=[
            pltpu.VMEM((b_per_w,), jnp.int32),
            pltpu.VMEM((b_per_w, embed), jnp.float32),
            pltpu.VMEM((b_per_w,), jnp.int32),
            pltpu.VMEM((b_per_w, embed), jnp.float32),
            pltpu.SemaphoreType.DMA,
            pltpu.SemaphoreType.DMA,
            pltpu.SemaphoreType.DMA,
            pltpu.SemaphoreType.DMA,
        ],
    )
    def gather_kernel(step_idx_hbm, label_idx_hbm, step_tab_hbm, label_tab_hbm,
                      srow_hbm, lrow_hbm,
                      sidx_v, srow_v, lidx_v, lrow_v, isem, ssem, lsem, osem):
        wid = lax.axis_index("s") * num_cores + lax.axis_index("c")
        base = wid * b_per_w
        icp1 = pltpu.async_copy(step_idx_hbm.at[pl.ds(base, b_per_w)], sidx_v, isem)
        icp2 = pltpu.async_copy(label_idx_hbm.at[pl.ds(base, b_per_w)], lidx_v, isem)
        icp1.wait()
        scp = pltpu.async_copy(step_tab_hbm.at[sidx_v], srow_v, ssem)
        icp2.wait()
        lcp = pltpu.async_copy(label_tab_hbm.at[lidx_v], lrow_v, lsem)
        scp.wait()
        ocp1 = pltpu.async_copy(srow_v, srow_hbm.at[pl.ds(base, b_per_w)], osem)
        lcp.wait()
        ocp2 = pltpu.async_copy(lrow_v, lrow_hbm.at[pl.ds(base, b_per_w)], osem)
        ocp1.wait()
        ocp2.wait()

    return gather_kernel(step_idx, label_idx, step_table, label_table)


def _add_body(x_ref, s_ref, l_ref, o_ref):
    emb = s_ref[...] + l_ref[...]
    o_ref[...] = x_ref[...] + emb[:, :, None]


def kernel(x, step, label, step_table, label_table):
    batch, embed, seq = x.shape
    srows, lrows = _gather_rows_sc(
        step.reshape(batch).astype(jnp.int32),
        label.reshape(batch).astype(jnp.int32),
        step_table, label_table)
    bt = 32
    return pl.pallas_call(
        _add_body,
        grid=(batch // bt,),
        in_specs=[
            pl.BlockSpec((bt, embed, seq), lambda i: (i, 0, 0)),
            pl.BlockSpec((bt, embed), lambda i: (i, 0)),
            pl.BlockSpec((bt, embed), lambda i: (i, 0)),
        ],
        out_specs=pl.BlockSpec((bt, embed, seq), lambda i: (i, 0, 0)),
        out_shape=jax.ShapeDtypeStruct((batch, embed, seq), jnp.float32),
    )(x, srows, lrows)
